# baseline (speedup 1.0000x reference)
import functools

import numpy as np
import jax
import jax.numpy as jnp
from jax import lax
from jax.experimental import pallas as pl
from jax.experimental.pallas import tpu as pltpu
from jax.experimental.pallas import tpu_sc as plsc

_NC = 2
_NS = 16
_NW = _NC * _NS

_D = 128
_DSH = 9
_EB = 16
_FCH = 16
_HID = 4 * _D


_CH = 80


def _gather_stage(hn, src, dst):
    N, D = hn.shape
    E = src.shape[0]
    per_w = E // _NW
    n_ch = per_w // _CH
    assert per_w * _NW == E and n_ch * _CH == per_w
    mesh = plsc.VectorSubcoreMesh(core_axis_name="c", subcore_axis_name="s")

    n_pairs = (n_ch - 1) // 2
    has_tail = (n_ch % 2) == 1

    stage_rows = 1000
    NB = 4
    n_grp = (n_ch - 1) // NB
    n_tail = n_ch - n_grp * NB

    def k(hn_hbm, src_hbm, dst_hbm, hs_hbm, hd_hbm,
          idx_v, bufs, hn_sh, gsems, osems):
        sid = lax.axis_index("s")
        wid = sid * _NC + lax.axis_index("c")
        base = wid * per_w

        @pl.when(sid < N // stage_rows)
        def _stage():
            pltpu.sync_copy(hn_hbm.at[pl.ds(sid * stage_rows, stage_rows)],
                            hn_sh.at[pl.ds(sid * stage_rows, stage_rows)])

        plsc.subcore_barrier()

        def gather_chunk(c, b):
            return pltpu.async_copy(
                hn_sh.at[idx_v.at[pl.ds(c * _CH, _CH)]], bufs.at[b], gsems[b])

        def drain(ref_like, sem):
            pltpu.make_async_copy(hn_hbm.at[pl.ds(0, _CH)], ref_like,
                                  sem).wait()

        for idx_hbm, out_hbm in ((src_hbm, hs_hbm), (dst_hbm, hd_hbm)):
            pltpu.sync_copy(idx_hbm.at[pl.ds(base, per_w)], idx_v)
            for b in range(NB):
                gather_chunk(b, b)

            def body(i, _, out_hbm=out_hbm):
                c = NB * i
                for b in range(NB):
                    drain(bufs.at[b], gsems[b])
                    pltpu.async_copy(
                        bufs.at[b],
                        out_hbm.at[pl.ds(base + (c + b) * _CH, _CH)],
                        osems[b])
                for b in range(NB):
                    drain(bufs.at[b], osems[b])

                    @pl.when(c + NB + b < n_ch)
                    def _next(c=c, b=b):
                        gather_chunk(c + NB + b, b)
                return 0

            lax.fori_loop(0, n_grp, body, 0)
            for b in range(n_tail):
                drain(bufs.at[b], gsems[b])
                pltpu.sync_copy(
                    bufs.at[b],
                    out_hbm.at[pl.ds(base + (n_grp * NB + b) * _CH, _CH)])

    return pl.kernel(
        k,
        out_type=[jax.ShapeDtypeStruct((E, D), jnp.float32),
                  jax.ShapeDtypeStruct((E, D), jnp.float32)],
        mesh=mesh,
        scratch_types=[
            pltpu.VMEM((per_w,), jnp.int32),
            pltpu.VMEM((NB, _CH, D), jnp.float32),
            pltpu.VMEM_SHARED((N, D), jnp.float32),
            [pltpu.SemaphoreType.DMA] * NB,
            [pltpu.SemaphoreType.DMA] * NB,
        ],
    )(hn, src, dst)


def _scatter_stage(scaled, dst3, zeros, N_pad):
    E, D = scaled.shape
    per_w = E // _NW
    n_ch = per_w // _CH
    rows_per_tile = N_pad // _NS
    assert rows_per_tile * _NS == N_pad and rows_per_tile % 8 == 0
    mesh = plsc.VectorSubcoreMesh(core_axis_name="c", subcore_axis_name="s")

    NB = 3
    n_grp = (n_ch - 1) // NB
    n_tail = n_ch - n_grp * NB

    @functools.partial(
        pl.kernel, mesh=mesh,
        out_type=jax.ShapeDtypeStruct((_NC, N_pad, D), jnp.float32),
        scratch_types=[
            pltpu.VMEM((n_ch, _CH), jnp.int32),
            pltpu.VMEM((NB, _CH, D), jnp.float32),
            pltpu.VMEM_SHARED((N_pad, D), jnp.float32),
            [pltpu.SemaphoreType.DMA] * NB,
            [pltpu.SemaphoreType.DMA] * NB,
        ],
    )
    def k(scaled_hbm, dst3_hbm, zeros_hbm, out_hbm,
          idx_v, bufs, acc_sh, lsems, ssems):
        cid = lax.axis_index("c")
        sid = lax.axis_index("s")
        wid = sid * _NC + cid
        base = wid * per_w
        row0 = sid * rows_per_tile
        pltpu.sync_copy(zeros_hbm, acc_sh.at[pl.ds(row0, rows_per_tile)])
        pltpu.sync_copy(dst3_hbm.at[wid], idx_v)
        plsc.subcore_barrier()

        def load_chunk(c, b):
            pltpu.async_copy(scaled_hbm.at[pl.ds(base + c * _CH, _CH)],
                             bufs.at[b], lsems[b])

        def drain(ref_like, sem):
            pltpu.make_async_copy(scaled_hbm.at[pl.ds(0, _CH)], ref_like,
                                  sem).wait()

        for b in range(NB):
            load_chunk(b, b)

        def body(i, _):
            c = NB * i
            for b in range(NB):
                drain(bufs.at[b], lsems[b])
                pltpu.async_copy(bufs.at[b], acc_sh.at[idx_v.at[c + b]],
                                 ssems[b], add=True)
            for b in range(NB):
                drain(bufs.at[b], ssems[b])

                @pl.when(c + NB + b < n_ch)
                def _next(c=c, b=b):
                    load_chunk(c + NB + b, b)
            return 0

        lax.fori_loop(0, n_grp, body, 0)
        for b in range(n_tail):
            drain(bufs.at[b], lsems[b])
            pltpu.sync_copy(bufs.at[b],
                            acc_sh.at[idx_v.at[n_grp * NB + b]], add=True)
        plsc.subcore_barrier()
        pltpu.sync_copy(acc_sh.at[pl.ds(row0, rows_per_tile)],
                        out_hbm.at[cid, pl.ds(row0, rows_per_tile)])

    return k(scaled, dst3, zeros)


def _edge_block_kernel(he_ref, hs_ref, hd_ref, fe_ref, fes_ref, norm_ref,
                       We1_ref, be1_ref, We2_ref, Wcomb_ref, Wf2s_ref,
                       Wu1_ref, bu1_ref, Wu2_ref,
                       he_out_ref, scaled_ref):
    bf = jnp.bfloat16
    he = he_ref[...]
    hs16 = hs_ref[...].astype(bf)
    hd16 = hd_ref[...].astype(bf)
    x = jnp.concatenate([he.astype(bf), hs16, hd16], axis=1)
    a = jnp.dot(x, We1_ref[...].astype(bf),
                preferred_element_type=jnp.float32) + be1_ref[...]
    ha = 0.5 * a
    av = (ha * (1.0 + jnp.tanh(ha))).astype(bf)
    v = jnp.dot(av, We2_ref[...].astype(bf),
                preferred_element_type=jnp.float32)
    pre = jnp.concatenate([fes_ref[...], v * fe_ref[...]],
                          axis=1).astype(bf)
    g = jnp.dot(pre, Wcomb_ref[...].astype(bf),
                preferred_element_type=jnp.float32)
    r = jnp.maximum(g[:, :_FCH] * (1.0 / np.sqrt(_EB)), 0.0)
    d48 = g[:, _FCH:]
    rd = (jnp.concatenate([r, r, r], axis=1) * d48).astype(bf)
    tp = jnp.dot(rd, Wf2s_ref[...].astype(bf),
                 preferred_element_type=jnp.float32)
    u = jnp.concatenate([tp.astype(bf), hs16, hd16], axis=1)
    b = jnp.dot(u, Wu1_ref[...].astype(bf),
                preferred_element_type=jnp.float32) + bu1_ref[...]
    hb = 0.5 * b
    bv = (hb * (1.0 + jnp.tanh(hb))).astype(bf)
    he_new = he + jnp.dot(bv, Wu2_ref[...].astype(bf),
                          preferred_element_type=jnp.float32)
    he_out_ref[...] = he_new
    scaled_ref[...] = he_new * norm_ref[...]


def _edge_stage(he, hs, hd, fe, fes, norm2d, We1, be1, We2, Wcomb, Wf2s,
                Wu1, bu1, Wu2):
    E = he.shape[0]
    BE = 1280 if E % 1280 == 0 else 512
    grid = (E // BE,)
    full = lambda shape: pl.BlockSpec(shape, lambda i: (0, 0))
    blk = lambda w: pl.BlockSpec((BE, w), lambda i: (i, 0))
    return pl.pallas_call(
        _edge_block_kernel,
        grid=grid,
        in_specs=[
            blk(_D), blk(_D), blk(_D), blk(_DSH), blk(_EB), blk(1),
            full(We1.shape), full((1, _HID)), full(We2.shape),
            full(Wcomb.shape), full(Wf2s.shape),
            full(Wu1.shape), full((1, _HID)), full(Wu2.shape),
        ],
        out_specs=[blk(_D), blk(_D)],
        out_shape=[
            jax.ShapeDtypeStruct((E, _D), jnp.float32),
            jax.ShapeDtypeStruct((E, _D), jnp.float32),
        ],
    )(he, hs, hd, fe, fes, norm2d, We1, be1.reshape(1, -1), We2, Wcomb, Wf2s,
      Wu1, bu1.reshape(1, -1), Wu2)


def _node_block_kernel(hn_ref, p0_ref, p1_ref, Wn1_ref, bn1_ref, Wn2_ref,
                       out_ref):
    hn = hn_ref[...]
    x = jnp.concatenate([hn, p0_ref[0] + p1_ref[0]], axis=1)
    a = jnp.dot(x, Wn1_ref[...], preferred_element_type=jnp.float32) + bn1_ref[...]
    ha = 0.5 * a
    av = ha * (1.0 + jnp.tanh(ha))
    out_ref[...] = hn + jnp.dot(av, Wn2_ref[...], preferred_element_type=jnp.float32)


def _node_stage(hn, partials, Wn1, bn1, Wn2):
    N = hn.shape[0]
    BN = 2000 if N % 2000 == 0 else N
    grid = (N // BN,)
    full = lambda shape: pl.BlockSpec(shape, lambda i: (0, 0))
    blk = lambda w: pl.BlockSpec((BN, w), lambda i: (i, 0))
    pblk = lambda c: pl.BlockSpec((1, BN, _D), lambda i, c=c: (c, i, 0))
    return pl.pallas_call(
        _node_block_kernel,
        grid=grid,
        in_specs=[blk(_D), pblk(0), pblk(1), full(Wn1.shape),
                  full((1, _HID)), full(Wn2.shape)],
        out_specs=blk(_D),
        out_shape=jax.ShapeDtypeStruct((N, _D), jnp.float32),
    )(hn, partials, partials, Wn1, bn1.reshape(1, -1), Wn2)


def kernel(hn, he, fe, fes, norm, edge_index, We1, be1, We2, Wf1, Wf2,
           Wu1, bu1, Wu2, Wn1, bn1, Wn2):
    src = edge_index[0]
    dst = edge_index[1]
    S = np.zeros((_DSH, 3), dtype=np.float32)
    S[0, 0] = 1.0
    S[1:4, 1] = 1.0 / np.sqrt(3.0)
    S[4:9, 2] = 1.0 / np.sqrt(5.0)
    S = S / np.sqrt(3.0)
    S48 = np.repeat(S, _EB, axis=1)
    Wcomb = jnp.zeros((_EB + _DSH, _FCH + 3 * _EB), dtype=jnp.float32)
    Wcomb = Wcomb.at[:_EB, :_FCH].set(Wf1)
    Wcomb = Wcomb.at[_EB:, _FCH:].set(jnp.asarray(S48))
    Wf2s = jnp.concatenate(
        [Wf2[:, p * _D:(p + 1) * _D] for p in range(3)], axis=0) \
        * (1.0 / np.sqrt(_FCH))

    hs, hd = _gather_stage(hn, src, dst)

    he_new, scaled = _edge_stage(he, hs, hd, fe, fes, norm.reshape(-1, 1),
                                 We1, be1, We2, Wcomb, Wf2s, Wu1, bu1, Wu2)

    N = hn.shape[0]
    E = he.shape[0]
    N_pad = ((N + 8 * _NS - 1) // (8 * _NS)) * (8 * _NS)
    dst3 = dst.reshape(_NW, (E // _NW) // _CH, _CH)
    zeros = jnp.zeros((N_pad // _NS, _D), dtype=jnp.float32)
    partials = _scatter_stage(scaled, dst3, zeros, N_pad)

    hn_new = _node_stage(hn, partials, Wn1, bn1, Wn2)
    return hn_new, he_new


# speedup vs baseline: 1.0909x; 1.0909x over previous
import functools

import numpy as np
import jax
import jax.numpy as jnp
from jax import lax
from jax.experimental import pallas as pl
from jax.experimental.pallas import tpu as pltpu
from jax.experimental.pallas import tpu_sc as plsc

_NC = 2
_NS = 16
_NW = _NC * _NS

_D = 128
_DSH = 9
_EB = 16
_FCH = 16
_HID = 4 * _D


_CH = 80


def _gather_stage(hn, src, dst):
    N, D = hn.shape
    E = src.shape[0]
    per_w = E // _NW
    n_ch = per_w // _CH
    assert per_w * _NW == E and n_ch * _CH == per_w
    mesh = plsc.VectorSubcoreMesh(core_axis_name="c", subcore_axis_name="s")

    n_pairs = (n_ch - 1) // 2
    has_tail = (n_ch % 2) == 1

    stage_rows = 1000
    NB = 4
    n_grp = (n_ch - 1) // NB
    n_tail = n_ch - n_grp * NB

    def k(hn_hbm, src_hbm, dst_hbm, hs_hbm, hd_hbm,
          idx_v, bufs, hn_sh, gsems, osems):
        sid = lax.axis_index("s")
        wid = sid * _NC + lax.axis_index("c")
        base = wid * per_w

        @pl.when(sid < N // stage_rows)
        def _stage():
            pltpu.sync_copy(hn_hbm.at[pl.ds(sid * stage_rows, stage_rows)],
                            hn_sh.at[pl.ds(sid * stage_rows, stage_rows)])

        plsc.subcore_barrier()

        def gather_chunk(c, b):
            return pltpu.async_copy(
                hn_sh.at[idx_v.at[pl.ds(c * _CH, _CH)]], bufs.at[b], gsems[b])

        def drain(ref_like, sem):
            pltpu.make_async_copy(hn_hbm.at[pl.ds(0, _CH)], ref_like,
                                  sem).wait()

        for idx_hbm, out_hbm in ((src_hbm, hs_hbm), (dst_hbm, hd_hbm)):
            pltpu.sync_copy(idx_hbm.at[pl.ds(base, per_w)], idx_v)
            for b in range(NB):
                gather_chunk(b, b)

            def body(i, _, out_hbm=out_hbm):
                c = NB * i
                for b in range(NB):
                    drain(bufs.at[b], gsems[b])
                    pltpu.async_copy(
                        bufs.at[b],
                        out_hbm.at[pl.ds(base + (c + b) * _CH, _CH)],
                        osems[b])
                for b in range(NB):
                    drain(bufs.at[b], osems[b])

                    @pl.when(c + NB + b < n_ch)
                    def _next(c=c, b=b):
                        gather_chunk(c + NB + b, b)
                return 0

            lax.fori_loop(0, n_grp, body, 0)
            for b in range(n_tail):
                drain(bufs.at[b], gsems[b])
                pltpu.sync_copy(
                    bufs.at[b],
                    out_hbm.at[pl.ds(base + (n_grp * NB + b) * _CH, _CH)])

    return pl.kernel(
        k,
        out_type=[jax.ShapeDtypeStruct((E, D), jnp.float32),
                  jax.ShapeDtypeStruct((E, D), jnp.float32)],
        mesh=mesh,
        scratch_types=[
            pltpu.VMEM((per_w,), jnp.int32),
            pltpu.VMEM((NB, _CH, D), jnp.float32),
            pltpu.VMEM_SHARED((N, D), jnp.float32),
            [pltpu.SemaphoreType.DMA] * NB,
            [pltpu.SemaphoreType.DMA] * NB,
        ],
    )(hn, src, dst)


def _scatter_stage(scaled, dst3, zeros, N_pad):
    E, D = scaled.shape
    per_w = E // _NW
    n_ch = per_w // _CH
    rows_per_tile = N_pad // _NS
    assert rows_per_tile * _NS == N_pad and rows_per_tile % 8 == 0
    mesh = plsc.VectorSubcoreMesh(core_axis_name="c", subcore_axis_name="s")

    NB = 3
    n_grp = (n_ch - 1) // NB
    n_tail = n_ch - n_grp * NB

    @functools.partial(
        pl.kernel, mesh=mesh,
        out_type=jax.ShapeDtypeStruct((_NC, N_pad, D), jnp.float32),
        scratch_types=[
            pltpu.VMEM((n_ch, _CH), jnp.int32),
            pltpu.VMEM((NB, _CH, D), jnp.float32),
            pltpu.VMEM_SHARED((N_pad, D), jnp.float32),
            [pltpu.SemaphoreType.DMA] * NB,
            [pltpu.SemaphoreType.DMA] * NB,
        ],
    )
    def k(scaled_hbm, dst3_hbm, zeros_hbm, out_hbm,
          idx_v, bufs, acc_sh, lsems, ssems):
        cid = lax.axis_index("c")
        sid = lax.axis_index("s")
        wid = sid * _NC + cid
        base = wid * per_w
        row0 = sid * rows_per_tile
        pltpu.sync_copy(zeros_hbm, acc_sh.at[pl.ds(row0, rows_per_tile)])
        pltpu.sync_copy(dst3_hbm.at[wid], idx_v)
        plsc.subcore_barrier()

        def load_chunk(c, b):
            pltpu.async_copy(scaled_hbm.at[pl.ds(base + c * _CH, _CH)],
                             bufs.at[b], lsems[b])

        def drain(ref_like, sem):
            pltpu.make_async_copy(scaled_hbm.at[pl.ds(0, _CH)], ref_like,
                                  sem).wait()

        for b in range(NB):
            load_chunk(b, b)

        def body(i, _):
            c = NB * i
            for b in range(NB):
                drain(bufs.at[b], lsems[b])
                pltpu.async_copy(bufs.at[b], acc_sh.at[idx_v.at[c + b]],
                                 ssems[b], add=True)
            for b in range(NB):
                drain(bufs.at[b], ssems[b])

                @pl.when(c + NB + b < n_ch)
                def _next(c=c, b=b):
                    load_chunk(c + NB + b, b)
            return 0

        lax.fori_loop(0, n_grp, body, 0)
        for b in range(n_tail):
            drain(bufs.at[b], lsems[b])
            pltpu.sync_copy(bufs.at[b],
                            acc_sh.at[idx_v.at[n_grp * NB + b]], add=True)
        plsc.subcore_barrier()
        pltpu.sync_copy(acc_sh.at[pl.ds(row0, rows_per_tile)],
                        out_hbm.at[cid, pl.ds(row0, rows_per_tile)])

    return k(scaled, dst3, zeros)


def _edge_block_kernel(he_ref, hs_ref, hd_ref, fe_ref, fes_ref, norm_ref,
                       We1_ref, be1_ref, We2_ref, Wcomb_ref, Wf2s_ref,
                       Wu1_ref, bu1_ref, Wu2_ref,
                       he_out_ref, scaled_ref):
    bf = jnp.bfloat16
    he = he_ref[...]
    hs16 = hs_ref[...].astype(bf)
    hd16 = hd_ref[...].astype(bf)
    x = jnp.concatenate([he.astype(bf), hs16, hd16], axis=1)
    a = jnp.dot(x, We1_ref[...].astype(bf),
                preferred_element_type=jnp.float32) + be1_ref[...]
    ha = 0.5 * a
    av = (ha * (1.0 + jnp.tanh(ha))).astype(bf)
    v = jnp.dot(av, We2_ref[...].astype(bf),
                preferred_element_type=jnp.float32)
    pre = jnp.concatenate([fes_ref[...], v * fe_ref[...]],
                          axis=1).astype(bf)
    g = jnp.dot(pre, Wcomb_ref[...].astype(bf),
                preferred_element_type=jnp.float32)
    r = jnp.maximum(g[:, :_FCH] * (1.0 / np.sqrt(_EB)), 0.0)
    d48 = g[:, _FCH:]
    rd = (jnp.concatenate([r, r, r], axis=1) * d48).astype(bf)
    tp = jnp.dot(rd, Wf2s_ref[...].astype(bf),
                 preferred_element_type=jnp.float32)
    u = jnp.concatenate([tp.astype(bf), hs16, hd16], axis=1)
    b = jnp.dot(u, Wu1_ref[...].astype(bf),
                preferred_element_type=jnp.float32) + bu1_ref[...]
    hb = 0.5 * b
    bv = (hb * (1.0 + jnp.tanh(hb))).astype(bf)
    he_new = he + jnp.dot(bv, Wu2_ref[...].astype(bf),
                          preferred_element_type=jnp.float32)
    he_out_ref[...] = he_new
    scaled_ref[...] = he_new * norm_ref[...]


def _edge_stage(he, hs, hd, fe, fes, norm2d, We1, be1, We2, Wcomb, Wf2s,
                Wu1, bu1, Wu2):
    E = he.shape[0]
    BE = 3200 if E % 3200 == 0 else 512
    grid = (E // BE,)
    full = lambda shape: pl.BlockSpec(shape, lambda i: (0, 0))
    blk = lambda w: pl.BlockSpec((BE, w), lambda i: (i, 0))
    return pl.pallas_call(
        _edge_block_kernel,
        grid=grid,
        in_specs=[
            blk(_D), blk(_D), blk(_D), blk(_DSH), blk(_EB), blk(1),
            full(We1.shape), full((1, _HID)), full(We2.shape),
            full(Wcomb.shape), full(Wf2s.shape),
            full(Wu1.shape), full((1, _HID)), full(Wu2.shape),
        ],
        out_specs=[blk(_D), blk(_D)],
        out_shape=[
            jax.ShapeDtypeStruct((E, _D), jnp.float32),
            jax.ShapeDtypeStruct((E, _D), jnp.float32),
        ],
    )(he, hs, hd, fe, fes, norm2d, We1, be1.reshape(1, -1), We2, Wcomb, Wf2s,
      Wu1, bu1.reshape(1, -1), Wu2)


def _node_block_kernel(hn_ref, p0_ref, p1_ref, Wn1_ref, bn1_ref, Wn2_ref,
                       out_ref):
    hn = hn_ref[...]
    x = jnp.concatenate([hn, p0_ref[0] + p1_ref[0]], axis=1)
    a = jnp.dot(x, Wn1_ref[...], preferred_element_type=jnp.float32) + bn1_ref[...]
    ha = 0.5 * a
    av = ha * (1.0 + jnp.tanh(ha))
    out_ref[...] = hn + jnp.dot(av, Wn2_ref[...], preferred_element_type=jnp.float32)


def _node_stage(hn, partials, Wn1, bn1, Wn2):
    N = hn.shape[0]
    BN = 2000 if N % 2000 == 0 else N
    grid = (N // BN,)
    full = lambda shape: pl.BlockSpec(shape, lambda i: (0, 0))
    blk = lambda w: pl.BlockSpec((BN, w), lambda i: (i, 0))
    pblk = lambda c: pl.BlockSpec((1, BN, _D), lambda i, c=c: (c, i, 0))
    return pl.pallas_call(
        _node_block_kernel,
        grid=grid,
        in_specs=[blk(_D), pblk(0), pblk(1), full(Wn1.shape),
                  full((1, _HID)), full(Wn2.shape)],
        out_specs=blk(_D),
        out_shape=jax.ShapeDtypeStruct((N, _D), jnp.float32),
    )(hn, partials, partials, Wn1, bn1.reshape(1, -1), Wn2)


def kernel(hn, he, fe, fes, norm, edge_index, We1, be1, We2, Wf1, Wf2,
           Wu1, bu1, Wu2, Wn1, bn1, Wn2):
    src = edge_index[0]
    dst = edge_index[1]
    S = np.zeros((_DSH, 3), dtype=np.float32)
    S[0, 0] = 1.0
    S[1:4, 1] = 1.0 / np.sqrt(3.0)
    S[4:9, 2] = 1.0 / np.sqrt(5.0)
    S = S / np.sqrt(3.0)
    S48 = np.repeat(S, _EB, axis=1)
    Wcomb = jnp.zeros((_EB + _DSH, _FCH + 3 * _EB), dtype=jnp.float32)
    Wcomb = Wcomb.at[:_EB, :_FCH].set(Wf1)
    Wcomb = Wcomb.at[_EB:, _FCH:].set(jnp.asarray(S48))
    Wf2s = jnp.concatenate(
        [Wf2[:, p * _D:(p + 1) * _D] for p in range(3)], axis=0) \
        * (1.0 / np.sqrt(_FCH))

    hs, hd = _gather_stage(hn, src, dst)

    he_new, scaled = _edge_stage(he, hs, hd, fe, fes, norm.reshape(-1, 1),
                                 We1, be1, We2, Wcomb, Wf2s, Wu1, bu1, Wu2)

    N = hn.shape[0]
    E = he.shape[0]
    N_pad = ((N + 8 * _NS - 1) // (8 * _NS)) * (8 * _NS)
    dst3 = dst.reshape(_NW, (E // _NW) // _CH, _CH)
    zeros = jnp.zeros((N_pad // _NS, _D), dtype=jnp.float32)
    partials = _scatter_stage(scaled, dst3, zeros, N_pad)

    hn_new = _node_stage(hn, partials, Wn1, bn1, Wn2)
    return hn_new, he_new


# speedup vs baseline: 1.0959x; 1.0046x over previous
import functools

import numpy as np
import jax
import jax.numpy as jnp
from jax import lax
from jax.experimental import pallas as pl
from jax.experimental.pallas import tpu as pltpu
from jax.experimental.pallas import tpu_sc as plsc

_NC = 2
_NS = 16
_NW = _NC * _NS

_D = 128
_DSH = 9
_EB = 16
_FCH = 16
_HID = 4 * _D


_CH = 80


def _gather_stage(hn, src, dst):
    N, D = hn.shape
    E = src.shape[0]
    per_w = E // _NW
    n_ch = per_w // _CH
    assert per_w * _NW == E and n_ch * _CH == per_w
    mesh = plsc.VectorSubcoreMesh(core_axis_name="c", subcore_axis_name="s")

    n_pairs = (n_ch - 1) // 2
    has_tail = (n_ch % 2) == 1

    stage_rows = 1000
    NB = 4
    n_grp = (n_ch - 1) // NB
    n_tail = n_ch - n_grp * NB

    def k(hn_hbm, src_hbm, dst_hbm, hs_hbm, hd_hbm,
          idx_v, bufs, hn_sh, gsems, osems):
        sid = lax.axis_index("s")
        wid = sid * _NC + lax.axis_index("c")
        base = wid * per_w

        @pl.when(sid < N // stage_rows)
        def _stage():
            pltpu.sync_copy(hn_hbm.at[pl.ds(sid * stage_rows, stage_rows)],
                            hn_sh.at[pl.ds(sid * stage_rows, stage_rows)])

        plsc.subcore_barrier()

        def gather_chunk(c, b):
            return pltpu.async_copy(
                hn_sh.at[idx_v.at[pl.ds(c * _CH, _CH)]], bufs.at[b], gsems[b])

        def drain(ref_like, sem):
            pltpu.make_async_copy(hn_hbm.at[pl.ds(0, _CH)], ref_like,
                                  sem).wait()

        for idx_hbm, out_hbm in ((src_hbm, hs_hbm), (dst_hbm, hd_hbm)):
            pltpu.sync_copy(idx_hbm.at[pl.ds(base, per_w)], idx_v)
            for b in range(NB):
                gather_chunk(b, b)

            def body(i, _, out_hbm=out_hbm):
                c = NB * i
                for b in range(NB):
                    drain(bufs.at[b], gsems[b])
                    pltpu.async_copy(
                        bufs.at[b],
                        out_hbm.at[pl.ds(base + (c + b) * _CH, _CH)],
                        osems[b])
                for b in range(NB):
                    drain(bufs.at[b], osems[b])

                    @pl.when(c + NB + b < n_ch)
                    def _next(c=c, b=b):
                        gather_chunk(c + NB + b, b)
                return 0

            lax.fori_loop(0, n_grp, body, 0)
            for b in range(n_tail):
                drain(bufs.at[b], gsems[b])
                pltpu.sync_copy(
                    bufs.at[b],
                    out_hbm.at[pl.ds(base + (n_grp * NB + b) * _CH, _CH)])

    return pl.kernel(
        k,
        out_type=[jax.ShapeDtypeStruct((E, D), jnp.float32),
                  jax.ShapeDtypeStruct((E, D), jnp.float32)],
        mesh=mesh,
        scratch_types=[
            pltpu.VMEM((per_w,), jnp.int32),
            pltpu.VMEM((NB, _CH, D), jnp.float32),
            pltpu.VMEM_SHARED((N, D), jnp.float32),
            [pltpu.SemaphoreType.DMA] * NB,
            [pltpu.SemaphoreType.DMA] * NB,
        ],
    )(hn, src, dst)


def _scatter_stage(scaled, dst3, zeros, N_pad):
    E, D = scaled.shape
    per_w = E // _NW
    n_ch = per_w // _CH
    rows_per_tile = N_pad // _NS
    assert rows_per_tile * _NS == N_pad and rows_per_tile % 8 == 0
    mesh = plsc.VectorSubcoreMesh(core_axis_name="c", subcore_axis_name="s")

    NB = 3
    n_grp = (n_ch - 1) // NB
    n_tail = n_ch - n_grp * NB

    @functools.partial(
        pl.kernel, mesh=mesh,
        out_type=jax.ShapeDtypeStruct((_NC, N_pad, D), jnp.float32),
        scratch_types=[
            pltpu.VMEM((n_ch, _CH), jnp.int32),
            pltpu.VMEM((NB, _CH, D), jnp.float32),
            pltpu.VMEM_SHARED((N_pad, D), jnp.float32),
            [pltpu.SemaphoreType.DMA] * NB,
            [pltpu.SemaphoreType.DMA] * NB,
        ],
    )
    def k(scaled_hbm, dst3_hbm, zeros_hbm, out_hbm,
          idx_v, bufs, acc_sh, lsems, ssems):
        cid = lax.axis_index("c")
        sid = lax.axis_index("s")
        wid = sid * _NC + cid
        base = wid * per_w
        row0 = sid * rows_per_tile
        pltpu.sync_copy(zeros_hbm, acc_sh.at[pl.ds(row0, rows_per_tile)])
        pltpu.sync_copy(dst3_hbm.at[wid], idx_v)
        plsc.subcore_barrier()

        def load_chunk(c, b):
            pltpu.async_copy(scaled_hbm.at[pl.ds(base + c * _CH, _CH)],
                             bufs.at[b], lsems[b])

        def drain(ref_like, sem):
            pltpu.make_async_copy(scaled_hbm.at[pl.ds(0, _CH)], ref_like,
                                  sem).wait()

        for b in range(NB):
            load_chunk(b, b)

        def body(i, _):
            c = NB * i
            for b in range(NB):
                drain(bufs.at[b], lsems[b])
                pltpu.async_copy(bufs.at[b], acc_sh.at[idx_v.at[c + b]],
                                 ssems[b], add=True)
            for b in range(NB):
                drain(bufs.at[b], ssems[b])

                @pl.when(c + NB + b < n_ch)
                def _next(c=c, b=b):
                    load_chunk(c + NB + b, b)
            return 0

        lax.fori_loop(0, n_grp, body, 0)
        for b in range(n_tail):
            drain(bufs.at[b], lsems[b])
            pltpu.sync_copy(bufs.at[b],
                            acc_sh.at[idx_v.at[n_grp * NB + b]], add=True)
        plsc.subcore_barrier()
        pltpu.sync_copy(acc_sh.at[pl.ds(row0, rows_per_tile)],
                        out_hbm.at[cid, pl.ds(row0, rows_per_tile)])

    return k(scaled, dst3, zeros)


def _edge_block_kernel(he_ref, hs_ref, hd_ref, fe_ref, fes_ref, norm_ref,
                       We1_ref, be1_ref, We2_ref, Wcomb_ref, Wf2s_ref,
                       Wu1_ref, bu1_ref, Wu2_ref,
                       he_out_ref, scaled_ref):
    bf = jnp.bfloat16
    he = he_ref[...]
    hs16 = hs_ref[...].astype(bf)
    hd16 = hd_ref[...].astype(bf)
    x = jnp.concatenate([he.astype(bf), hs16, hd16], axis=1)
    a = jnp.dot(x, We1_ref[...].astype(bf),
                preferred_element_type=jnp.float32) + be1_ref[...]
    ha = 0.5 * a
    av = (ha * (1.0 + jnp.tanh(ha))).astype(bf)
    v = jnp.dot(av, We2_ref[...].astype(bf),
                preferred_element_type=jnp.float32)
    pre = jnp.concatenate([fes_ref[...], v * fe_ref[...]],
                          axis=1).astype(bf)
    g = jnp.dot(pre, Wcomb_ref[...].astype(bf),
                preferred_element_type=jnp.float32)
    r = jnp.maximum(g[:, :_FCH] * (1.0 / np.sqrt(_EB)), 0.0)
    d48 = g[:, _FCH:]
    rd = (jnp.concatenate([r, r, r], axis=1) * d48).astype(bf)
    tp = jnp.dot(rd, Wf2s_ref[...].astype(bf),
                 preferred_element_type=jnp.float32)
    u = jnp.concatenate([tp.astype(bf), hs16, hd16], axis=1)
    b = jnp.dot(u, Wu1_ref[...].astype(bf),
                preferred_element_type=jnp.float32) + bu1_ref[...]
    hb = 0.5 * b
    bv = (hb * (1.0 + jnp.tanh(hb))).astype(bf)
    he_new = he + jnp.dot(bv, Wu2_ref[...].astype(bf),
                          preferred_element_type=jnp.float32)
    he_out_ref[...] = he_new
    scaled_ref[...] = he_new * norm_ref[...]


def _edge_stage(he, hs, hd, fe, fes, norm2d, We1, be1, We2, Wcomb, Wf2s,
                Wu1, bu1, Wu2):
    E = he.shape[0]
    BE = 4000 if E % 4000 == 0 else 512
    grid = (E // BE,)
    full = lambda shape: pl.BlockSpec(shape, lambda i: (0, 0))
    blk = lambda w: pl.BlockSpec((BE, w), lambda i: (i, 0))
    return pl.pallas_call(
        _edge_block_kernel,
        grid=grid,
        in_specs=[
            blk(_D), blk(_D), blk(_D), blk(_DSH), blk(_EB), blk(1),
            full(We1.shape), full((1, _HID)), full(We2.shape),
            full(Wcomb.shape), full(Wf2s.shape),
            full(Wu1.shape), full((1, _HID)), full(Wu2.shape),
        ],
        out_specs=[blk(_D), blk(_D)],
        out_shape=[
            jax.ShapeDtypeStruct((E, _D), jnp.float32),
            jax.ShapeDtypeStruct((E, _D), jnp.float32),
        ],
    )(he, hs, hd, fe, fes, norm2d, We1, be1.reshape(1, -1), We2, Wcomb, Wf2s,
      Wu1, bu1.reshape(1, -1), Wu2)


def _node_block_kernel(hn_ref, p0_ref, p1_ref, Wn1_ref, bn1_ref, Wn2_ref,
                       out_ref):
    hn = hn_ref[...]
    x = jnp.concatenate([hn, p0_ref[0] + p1_ref[0]], axis=1)
    a = jnp.dot(x, Wn1_ref[...], preferred_element_type=jnp.float32) + bn1_ref[...]
    ha = 0.5 * a
    av = ha * (1.0 + jnp.tanh(ha))
    out_ref[...] = hn + jnp.dot(av, Wn2_ref[...], preferred_element_type=jnp.float32)


def _node_stage(hn, partials, Wn1, bn1, Wn2):
    N = hn.shape[0]
    BN = 2000 if N % 2000 == 0 else N
    grid = (N // BN,)
    full = lambda shape: pl.BlockSpec(shape, lambda i: (0, 0))
    blk = lambda w: pl.BlockSpec((BN, w), lambda i: (i, 0))
    pblk = lambda c: pl.BlockSpec((1, BN, _D), lambda i, c=c: (c, i, 0))
    return pl.pallas_call(
        _node_block_kernel,
        grid=grid,
        in_specs=[blk(_D), pblk(0), pblk(1), full(Wn1.shape),
                  full((1, _HID)), full(Wn2.shape)],
        out_specs=blk(_D),
        out_shape=jax.ShapeDtypeStruct((N, _D), jnp.float32),
    )(hn, partials, partials, Wn1, bn1.reshape(1, -1), Wn2)


def kernel(hn, he, fe, fes, norm, edge_index, We1, be1, We2, Wf1, Wf2,
           Wu1, bu1, Wu2, Wn1, bn1, Wn2):
    src = edge_index[0]
    dst = edge_index[1]
    S = np.zeros((_DSH, 3), dtype=np.float32)
    S[0, 0] = 1.0
    S[1:4, 1] = 1.0 / np.sqrt(3.0)
    S[4:9, 2] = 1.0 / np.sqrt(5.0)
    S = S / np.sqrt(3.0)
    S48 = np.repeat(S, _EB, axis=1)
    Wcomb = jnp.zeros((_EB + _DSH, _FCH + 3 * _EB), dtype=jnp.float32)
    Wcomb = Wcomb.at[:_EB, :_FCH].set(Wf1)
    Wcomb = Wcomb.at[_EB:, _FCH:].set(jnp.asarray(S48))
    Wf2s = jnp.concatenate(
        [Wf2[:, p * _D:(p + 1) * _D] for p in range(3)], axis=0) \
        * (1.0 / np.sqrt(_FCH))

    hs, hd = _gather_stage(hn, src, dst)

    he_new, scaled = _edge_stage(he, hs, hd, fe, fes, norm.reshape(-1, 1),
                                 We1, be1, We2, Wcomb, Wf2s, Wu1, bu1, Wu2)

    N = hn.shape[0]
    E = he.shape[0]
    N_pad = ((N + 8 * _NS - 1) // (8 * _NS)) * (8 * _NS)
    dst3 = dst.reshape(_NW, (E // _NW) // _CH, _CH)
    zeros = jnp.zeros((N_pad // _NS, _D), dtype=jnp.float32)
    partials = _scatter_stage(scaled, dst3, zeros, N_pad)

    hn_new = _node_stage(hn, partials, Wn1, bn1, Wn2)
    return hn_new, he_new
